# Initial kernel scaffold; baseline (speedup 1.0000x reference)
import jax
import jax.numpy as jnp
from jax import lax
from jax.experimental import pallas as pl
from jax.experimental.pallas import tpu as pltpu
from jax.experimental.pallas import tpu_sc as plsc

N = 10000
E = 320000
D = 128
W = 144
NCLS = 64

NC = 2
NS = 16
NW = NC * NS
EPT = E // NW
K = 80
CHUNKS = EPT // K
RPT = N // NS


_R = 1000


def _fc1_table_body(x_ref, w1_ref, b1_ref, out_ref):
    x = lax.dot_general(x_ref[...], w1_ref[...],
                        dimension_numbers=(((1,), (1,)), ((), ())),
                        preferred_element_type=jnp.float32)
    x = jnp.maximum(x + b1_ref[...], 0.0)
    nc = jnp.maximum(jnp.sqrt(jnp.sum(x * x, axis=1, keepdims=True)), 1e-12)
    out_ref[:, 0:D] = x / nc
    cols = lax.broadcasted_iota(jnp.int32, (_R, W - D), 1)
    out_ref[:, D:W] = jnp.where(cols == 0, nc, 0.0)


def _fc1_table(x, w1, b1):
    return pl.pallas_call(
        _fc1_table_body,
        grid=(N // _R,),
        in_specs=[
            pl.BlockSpec((_R, D), lambda i: (i, 0)),
            pl.BlockSpec((D, D), lambda i: (0, 0)),
            pl.BlockSpec((D,), lambda i: (0,)),
        ],
        out_specs=pl.BlockSpec((_R, W), lambda i: (i, 0)),
        out_shape=jax.ShapeDtypeStruct((N, W), jnp.float32),
    )(x, w1, b1)


def _combine_table_body(p_ref, out_ref):
    row = p_ref[0] + p_ref[1]
    s = row[:, D:D + 1]
    x1 = row[:, 0:D] / (s + 1e-12)
    nc = jnp.maximum(jnp.sqrt(jnp.sum(x1 * x1, axis=1, keepdims=True)), 1e-12)
    out_ref[:, 0:D] = x1 / nc
    cols = lax.broadcasted_iota(jnp.int32, (_R, W - D), 1)
    out_ref[:, D:W] = jnp.where(cols == 0, nc, 0.0)


def _combine_table(p):
    return pl.pallas_call(
        _combine_table_body,
        grid=(N // _R,),
        in_specs=[pl.BlockSpec((2, _R, W), lambda i: (0, i, 0))],
        out_specs=pl.BlockSpec((_R, W), lambda i: (i, 0)),
        out_shape=jax.ShapeDtypeStruct((N, W), jnp.float32),
    )(p)


def _final_body(p_ref, w2_ref, b2_ref, out_ref):
    row = p_ref[0] + p_ref[1]
    s = row[:, D:D + 1]
    x2 = row[:, 0:D] / (s + 1e-12)
    y = lax.dot_general(x2, w2_ref[...],
                        dimension_numbers=(((1,), (1,)), ((), ())),
                        preferred_element_type=jnp.float32)
    out_ref[...] = y + b2_ref[...]


def _final(p, w2, b2):
    return pl.pallas_call(
        _final_body,
        grid=(N // _R,),
        in_specs=[
            pl.BlockSpec((2, _R, W), lambda i: (0, i, 0)),
            pl.BlockSpec((NCLS, D), lambda i: (0, 0)),
            pl.BlockSpec((NCLS,), lambda i: (0,)),
        ],
        out_specs=pl.BlockSpec((_R, NCLS), lambda i: (i, 0)),
        out_shape=jax.ShapeDtypeStruct((N, NCLS), jnp.float32),
    )(p, w2, b2)


def _agnn_sc_body(table_hbm, src_hbm, dst_hbm, beta_hbm, out_hbm,
                  src_idx, dst_idx, srcrows, dstrows, msg,
                  kbuf, exbuf, beta_v, shared_out, gsem):
    c = lax.axis_index("c")
    s = lax.axis_index("s")
    wid = c * NS + s

    pltpu.sync_copy(src_hbm.at[wid], src_idx)
    pltpu.sync_copy(dst_hbm.at[wid], dst_idx)
    pltpu.sync_copy(beta_hbm, beta_v)

    zv = jnp.zeros((16,), jnp.float32)

    def zero_msg(r, carry):
        for cc in range(W // 16):
            msg[r, pl.ds(cc * 16, 16)] = zv
        return carry

    lax.fori_loop(0, K, zero_msg, 0)
    r0 = s * RPT
    for t in range(7):
        pltpu.sync_copy(msg, shared_out.at[pl.ds(r0 + t * K, K)])
    pltpu.sync_copy(msg.at[pl.ds(0, RPT - 7 * K)],
                    shared_out.at[pl.ds(r0 + 7 * K, RPT - 7 * K)])
    plsc.subcore_barrier()

    lanes = lax.iota(jnp.int32, 16)
    bvec = beta_v[...]

    def chunk_body(j, carry):
        d1 = pltpu.async_copy(table_hbm.at[src_idx.at[j]], srcrows, gsem)
        d2 = pltpu.async_copy(table_hbm.at[dst_idx.at[j]], dstrows, gsem)
        d1.wait()
        d2.wait()
        for g in range(K // 16):
            rows16 = g * 16 + lanes

            def dot_body(t, acc):
                for u in range(8):
                    col = jnp.full((16,), t * 8 + u, jnp.int32)
                    a = plsc.load_gather(srcrows, [rows16, col])
                    b = plsc.load_gather(dstrows, [rows16, col])
                    acc = acc + a * b
                return acc

            acc = lax.fori_loop(0, D // 8, dot_body, jnp.zeros((16,), jnp.float32))
            nrm = plsc.load_gather(srcrows, [rows16, jnp.full((16,), D, jnp.int32)])
            ex = jnp.exp(bvec * acc)
            kbuf[...] = ex * nrm
            exbuf[...] = ex
            for e in range(16):
                r = g * 16 + e
                ke = kbuf[e]
                for cc in range(D // 16):
                    msg[r, pl.ds(cc * 16, 16)] = ke * srcrows[r, pl.ds(cc * 16, 16)]
                msg[r, pl.ds(D, 16)] = jnp.where(lanes == 0, exbuf[e], 0.0)
        pltpu.sync_copy(msg, shared_out.at[dst_idx.at[j]], add=True)
        return carry

    lax.fori_loop(0, CHUNKS, chunk_body, 0)
    plsc.subcore_barrier()

    pltpu.sync_copy(shared_out.at[pl.ds(r0, RPT)],
                    out_hbm.at[c, pl.ds(r0, RPT)])


def _agnn_layer(table, src3, dst3, beta_arr):
    mesh = plsc.VectorSubcoreMesh(core_axis_name="c", subcore_axis_name="s",
                                  num_cores=NC, num_subcores=NS)
    f = pl.kernel(
        _agnn_sc_body,
        out_type=jax.ShapeDtypeStruct((NC, N, W), jnp.float32),
        mesh=mesh,
        scratch_types=[
            pltpu.VMEM((CHUNKS, K), jnp.int32),
            pltpu.VMEM((CHUNKS, K), jnp.int32),
            pltpu.VMEM((K, W), jnp.float32),
            pltpu.VMEM((K, W), jnp.float32),
            pltpu.VMEM((K, W), jnp.float32),
            pltpu.VMEM((16,), jnp.float32),
            pltpu.VMEM((16,), jnp.float32),
            pltpu.VMEM((16,), jnp.float32),
            pltpu.VMEM_SHARED((N, W), jnp.float32),
            pltpu.SemaphoreType.DMA,
        ],
    )
    return f(table, src3, dst3, beta_arr)


def kernel(input_features, edge_index, order_attn, W1, b1, beta1, beta2, W2, b2):
    src3 = edge_index[0].reshape(NW, CHUNKS, K)
    dst3 = edge_index[1].reshape(NW, CHUNKS, K)
    beta1_arr = jnp.full((16,), beta1, jnp.float32)
    beta2_arr = jnp.full((16,), beta2, jnp.float32)

    table0 = _fc1_table(input_features, W1, b1)
    p1 = _agnn_layer(table0, src3, dst3, beta1_arr)
    table1 = _combine_table(p1)
    p2 = _agnn_layer(table1, src3, dst3, beta2_arr)
    return _final(p2, W2, b2)


# speedup vs baseline: 6.1921x; 6.1921x over previous
import jax
import jax.numpy as jnp
from jax import lax
from jax.experimental import pallas as pl
from jax.experimental.pallas import tpu as pltpu
from jax.experimental.pallas import tpu_sc as plsc

N = 10000
E = 320000
D = 128
W = 144
NCLS = 64

NC = 2
NS = 16
NW = NC * NS
EPT = E // NW
K = 80
G = K // 16
CHUNKS = EPT // K

HALF0 = 5008
HALF1 = N - HALF0
ACC_ROWS = 5024
DUMP = 5016


_R = 1000


def _fc1_table_body(x_ref, w1_ref, b1_ref, out_ref):
    x = lax.dot_general(x_ref[...], w1_ref[...],
                        dimension_numbers=(((1,), (1,)), ((), ())),
                        preferred_element_type=jnp.float32)
    x = jnp.maximum(x + b1_ref[...], 0.0)
    nc = jnp.maximum(jnp.sqrt(jnp.sum(x * x, axis=1, keepdims=True)), 1e-12)
    out_ref[:, 0:D] = x / nc
    cols = lax.broadcasted_iota(jnp.int32, (_R, W - D), 1)
    out_ref[:, D:W] = jnp.where(cols == 0, nc, 0.0)


def _fc1_table(x, w1, b1):
    return pl.pallas_call(
        _fc1_table_body,
        grid=(N // _R,),
        in_specs=[
            pl.BlockSpec((_R, D), lambda i: (i, 0)),
            pl.BlockSpec((D, D), lambda i: (0, 0)),
            pl.BlockSpec((D,), lambda i: (0,)),
        ],
        out_specs=pl.BlockSpec((_R, W), lambda i: (i, 0)),
        out_shape=jax.ShapeDtypeStruct((N, W), jnp.float32),
    )(x, w1, b1)


def _combine_table_body(p_ref, out_ref):
    row = p_ref[0] + p_ref[1]
    s = row[:, D:D + 1]
    x1 = row[:, 0:D] / (s + 1e-12)
    nc = jnp.maximum(jnp.sqrt(jnp.sum(x1 * x1, axis=1, keepdims=True)), 1e-12)
    out_ref[:, 0:D] = x1 / nc
    cols = lax.broadcasted_iota(jnp.int32, (_R, W - D), 1)
    out_ref[:, D:W] = jnp.where(cols == 0, nc, 0.0)


def _combine_table(p):
    return pl.pallas_call(
        _combine_table_body,
        grid=(N // _R,),
        in_specs=[pl.BlockSpec((2, _R, W), lambda i: (0, i, 0))],
        out_specs=pl.BlockSpec((_R, W), lambda i: (i, 0)),
        out_shape=jax.ShapeDtypeStruct((N, W), jnp.float32),
    )(p)


def _final_body(p_ref, w2_ref, b2_ref, out_ref):
    row = p_ref[0] + p_ref[1]
    s = row[:, D:D + 1]
    x2 = row[:, 0:D] / (s + 1e-12)
    y = lax.dot_general(x2, w2_ref[...],
                        dimension_numbers=(((1,), (1,)), ((), ())),
                        preferred_element_type=jnp.float32)
    out_ref[...] = y + b2_ref[...]


def _final(p, w2, b2):
    return pl.pallas_call(
        _final_body,
        grid=(N // _R,),
        in_specs=[
            pl.BlockSpec((2, _R, W), lambda i: (0, i, 0)),
            pl.BlockSpec((NCLS, D), lambda i: (0, 0)),
            pl.BlockSpec((NCLS,), lambda i: (0,)),
        ],
        out_specs=pl.BlockSpec((_R, NCLS), lambda i: (i, 0)),
        out_shape=jax.ShapeDtypeStruct((N, NCLS), jnp.float32),
    )(p, w2, b2)


def _agnn_sc_body(table_hbm, src_hbm, dst_hbm, beta_hbm, out_hbm,
                  src_idx, dst_idx, srcrows, dstrows, msg, sidx,
                  exbuf, kvbuf, beta_v, acc_sh, gsem):
    c = lax.axis_index("c")
    s = lax.axis_index("s")
    wid = c * NS + s

    pltpu.sync_copy(src_hbm.at[wid], src_idx)
    pltpu.sync_copy(dst_hbm.at[wid], dst_idx)
    pltpu.sync_copy(beta_hbm, beta_v)

    zv = jnp.zeros((16,), jnp.float32)
    lanes = lax.iota(jnp.int32, 16)
    bvec = beta_v[...]

    def zero_msg(r, carry):
        for cc in range(W // 16):
            msg[r, pl.ds(cc * 16, 16)] = zv
        return carry

    def zero_acc():
        z0 = s * (ACC_ROWS // NS)
        for t in range(3):
            pltpu.sync_copy(msg, acc_sh.at[pl.ds(z0 + t * K, K)])
        pltpu.sync_copy(msg.at[pl.ds(0, 74)], acc_sh.at[pl.ds(z0 + 3 * K, 74)])

    def build_msg(kv, ex):
        for e in range(16):
            yield kv[e], ex[e]

    def make_pass(first):
        def chunk_body(j, carry):
            d1 = pltpu.async_copy(table_hbm.at[src_idx.at[j]], srcrows, gsem)
            if first:
                d2 = pltpu.async_copy(table_hbm.at[dst_idx.at[j]], dstrows, gsem)
            d1.wait()
            if first:
                d2.wait()
            for g in range(G):
                rows16 = g * 16 + lanes
                ebase = j * K + g * 16
                if first:
                    def dot_body(t, a0):
                        for u in range(8):
                            col = jnp.full((16,), t * 8 + u, jnp.int32)
                            a = plsc.load_gather(srcrows, [rows16, col])
                            b = plsc.load_gather(dstrows, [rows16, col])
                            a0 = a0 + a * b
                        return a0

                    acc = lax.fori_loop(0, D // 8, dot_body,
                                        jnp.zeros((16,), jnp.float32))
                    nrm = plsc.load_gather(
                        srcrows, [rows16, jnp.full((16,), D, jnp.int32)])
                    ex = jnp.exp(bvec * acc)
                    kv = ex * nrm
                    exbuf[pl.ds(ebase, 16)] = ex
                    kvbuf[pl.ds(ebase, 16)] = kv
                else:
                    ex = exbuf[pl.ds(ebase, 16)]
                    kv = kvbuf[pl.ds(ebase, 16)]
                for e in range(16):
                    r = g * 16 + e
                    ke = kv[e]
                    for cc in range(D // 16):
                        msg[r, pl.ds(cc * 16, 16)] = (
                            ke * srcrows[r, pl.ds(cc * 16, 16)])
                    msg[r, pl.ds(D, 16)] = jnp.where(lanes == 0, ex[e], 0.0)
                dv = dst_idx[j, pl.ds(g * 16, 16)]
                if first:
                    cidx = jnp.where(dv < HALF0, dv, DUMP)
                else:
                    cidx = jnp.where(dv >= HALF0, dv - HALF0, DUMP)
                sidx[pl.ds(g * 16, 16)] = cidx
            pltpu.sync_copy(msg, acc_sh.at[sidx], add=True)
            return carry
        return chunk_body

    lax.fori_loop(0, K, zero_msg, 0)
    zero_acc()
    plsc.subcore_barrier()
    lax.fori_loop(0, CHUNKS, make_pass(True), 0)
    plsc.subcore_barrier()
    pltpu.sync_copy(acc_sh.at[pl.ds(s * (HALF0 // NS), HALF0 // NS)],
                    out_hbm.at[c, pl.ds(s * (HALF0 // NS), HALF0 // NS)])
    plsc.subcore_barrier()

    lax.fori_loop(0, K, zero_msg, 0)
    zero_acc()
    plsc.subcore_barrier()
    lax.fori_loop(0, CHUNKS, make_pass(False), 0)
    plsc.subcore_barrier()
    pltpu.sync_copy(acc_sh.at[pl.ds(s * (HALF1 // NS), HALF1 // NS)],
                    out_hbm.at[c, pl.ds(HALF0 + s * (HALF1 // NS), HALF1 // NS)])


def _agnn_layer(table, src3, dst3, beta_arr):
    mesh = plsc.VectorSubcoreMesh(core_axis_name="c", subcore_axis_name="s",
                                  num_cores=NC, num_subcores=NS)
    f = pl.kernel(
        _agnn_sc_body,
        out_type=jax.ShapeDtypeStruct((NC, N, W), jnp.float32),
        mesh=mesh,
        scratch_types=[
            pltpu.VMEM((CHUNKS, K), jnp.int32),
            pltpu.VMEM((CHUNKS, K), jnp.int32),
            pltpu.VMEM((K, W), jnp.float32),
            pltpu.VMEM((K, W), jnp.float32),
            pltpu.VMEM((K, W), jnp.float32),
            pltpu.VMEM((K,), jnp.int32),
            pltpu.VMEM((EPT,), jnp.float32),
            pltpu.VMEM((EPT,), jnp.float32),
            pltpu.VMEM((16,), jnp.float32),
            pltpu.VMEM_SHARED((ACC_ROWS, W), jnp.float32),
            pltpu.SemaphoreType.DMA,
        ],
        compiler_params=pltpu.CompilerParams(use_tc_tiling_on_sc=False,
                                             needs_layout_passes=False),
    )
    return f(table, src3, dst3, beta_arr)


def kernel(input_features, edge_index, order_attn, W1, b1, beta1, beta2, W2, b2):
    src3 = edge_index[0].reshape(NW, CHUNKS, K)
    dst3 = edge_index[1].reshape(NW, CHUNKS, K)
    beta1_arr = jnp.full((16,), beta1, jnp.float32)
    beta2_arr = jnp.full((16,), beta2, jnp.float32)

    table0 = _fc1_table(input_features, W1, b1)
    p1 = _agnn_layer(table0, src3, dst3, beta1_arr)
    table1 = _combine_table(p1)
    p2 = _agnn_layer(table1, src3, dst3, beta2_arr)
    return _final(p2, W2, b2)
